# Initial kernel scaffold; baseline (speedup 1.0000x reference)
import jax
import jax.numpy as jnp
from jax.experimental import pallas as pl


def kernel(z, codebook):
    raise NotImplementedError("write your pallas kernel here")


# speedup vs baseline: 1.4345x; 1.4345x over previous
import functools

import jax
import jax.numpy as jnp
from jax import lax
from jax.experimental import pallas as pl
from jax.experimental.pallas import tpu as pltpu
from jax.experimental.pallas import tpu_sc as plsc

COMMIT = 0.25


def _encode_body(z_ref, cb_ref, idx_ref, loss_ref):
    b = pl.program_id(0)
    zb = z_ref[0]
    cb = cb_ref[...]
    T, D = zb.shape
    K = cb.shape[0]
    mm = lax.dot_general(zb, cb, (((1,), (1,)), ((), ())),
                         preferred_element_type=jnp.float32)
    znorm = jnp.sum(zb * zb, axis=1, keepdims=True)
    cnorm = jnp.sum(cb * cb, axis=1)
    d = znorm - 2.0 * mm + cnorm[None, :]
    mind = jnp.min(d, axis=1, keepdims=True)
    iota = lax.broadcasted_iota(jnp.int32, (T, K), 1)
    idx = jnp.min(jnp.where(d == mind, iota, K), axis=1)
    idx_ref[0, 0, :] = idx

    @pl.when(b == 0)
    def _():
        loss_ref[...] = jnp.zeros((1, 1), jnp.float32)

    loss_ref[...] += jnp.sum(mind).reshape(1, 1)


def _encode(z, codebook):
    B, T, D = z.shape
    K = codebook.shape[0]
    idx3, losssum = pl.pallas_call(
        _encode_body,
        grid=(B,),
        in_specs=[
            pl.BlockSpec((1, T, D), lambda i: (i, 0, 0)),
            pl.BlockSpec((K, D), lambda i: (0, 0)),
        ],
        out_specs=[
            pl.BlockSpec((1, 1, T), lambda i: (i, 0, 0)),
            pl.BlockSpec((1, 1), lambda i: (0, 0)),
        ],
        out_shape=[
            jax.ShapeDtypeStruct((B, 1, T), jnp.int32),
            jax.ShapeDtypeStruct((1, 1), jnp.float32),
        ],
    )(z, codebook)
    return idx3.reshape(B, T), losssum[0, 0]


_NC, _NS = 2, 16
_NW = _NC * _NS
_CHUNK = 96


def _sc_gather_body(n, per, cb_hbm, idx_hbm, out_hbm, idx_v, buf0, buf1,
                    sem0, sem1):
    wid = lax.axis_index("s") * _NC + lax.axis_index("c")
    base = wid * per
    pltpu.sync_copy(idx_hbm.at[pl.ds(base, per)], idx_v)
    bufs = (buf0, buf1)
    sems = (sem0, sem1)
    nch = per // _CHUNK
    copies = [None] * nch
    copies[0] = pltpu.async_copy(cb_hbm.at[idx_v.at[pl.ds(0, _CHUNK)]],
                                 bufs[0], sems[0])
    for j in range(nch):
        if j + 1 < nch:
            copies[j + 1] = pltpu.async_copy(
                cb_hbm.at[idx_v.at[pl.ds((j + 1) * _CHUNK, _CHUNK)]],
                bufs[(j + 1) % 2], sems[(j + 1) % 2])
        copies[j].wait()
        pltpu.sync_copy(bufs[j % 2],
                        out_hbm.at[pl.ds(base + j * _CHUNK, _CHUNK)])


def _sc_gather(codebook, idx_flat):
    n = idx_flat.shape[0]
    K, D = codebook.shape
    per = n // _NW
    mesh = plsc.VectorSubcoreMesh(core_axis_name="c", subcore_axis_name="s")
    fn = functools.partial(
        pl.kernel,
        out_type=jax.ShapeDtypeStruct((n, D), jnp.float32),
        mesh=mesh,
        scratch_types=[
            pltpu.VMEM((per,), jnp.int32),
            pltpu.VMEM((_CHUNK, D), jnp.float32),
            pltpu.VMEM((_CHUNK, D), jnp.float32),
            pltpu.SemaphoreType.DMA,
            pltpu.SemaphoreType.DMA,
        ],
    )(functools.partial(_sc_gather_body, n, per))
    return fn(codebook, idx_flat)


def kernel(z, codebook):
    B, T, D = z.shape
    idx, losssum = _encode(z, codebook)
    loss = losssum * ((1.0 + COMMIT) / (B * T * D))
    quant = _sc_gather(codebook, idx.reshape(-1))
    return quant.reshape(B, T, D), loss, idx


# speedup vs baseline: 1.6622x; 1.1587x over previous
import functools

import jax
import jax.numpy as jnp
from jax import lax
from jax.experimental import pallas as pl
from jax.experimental.pallas import tpu as pltpu
from jax.experimental.pallas import tpu_sc as plsc

COMMIT = 0.25


def _encode_body(z_ref, cb_ref, idx_ref, loss_ref):
    b = pl.program_id(0)
    zb = z_ref[0]
    cb = cb_ref[...]
    T, D = zb.shape
    K = cb.shape[0]
    mm = lax.dot_general(zb, cb, (((1,), (1,)), ((), ())),
                         preferred_element_type=jnp.float32)
    znorm = jnp.sum(zb * zb, axis=1, keepdims=True)
    cnorm = jnp.sum(cb * cb, axis=1)
    d = znorm - 2.0 * mm + cnorm[None, :]
    mind = jnp.min(d, axis=1, keepdims=True)
    iota = lax.broadcasted_iota(jnp.int32, (T, K), 1)
    idx = jnp.min(jnp.where(d == mind, iota, K), axis=1, keepdims=True)
    idx_ref[...] = idx.reshape(1, T, 1)

    @pl.when(b == 0)
    def _():
        loss_ref[...] = jnp.zeros((1, 1), jnp.float32)

    loss_ref[...] += jnp.sum(mind).reshape(1, 1)


def _encode(z, codebook):
    B, T, D = z.shape
    K = codebook.shape[0]
    idxT, losssum = pl.pallas_call(
        _encode_body,
        grid=(B,),
        in_specs=[
            pl.BlockSpec((1, T, D), lambda i: (i, 0, 0)),
            pl.BlockSpec((K, D), lambda i: (0, 0)),
        ],
        out_specs=[
            pl.BlockSpec((1, T, 1), lambda i: (i, 0, 0)),
            pl.BlockSpec((1, 1), lambda i: (0, 0)),
        ],
        out_shape=[
            jax.ShapeDtypeStruct((B, T, 1), jnp.int32),
            jax.ShapeDtypeStruct((1, 1), jnp.float32),
        ],
    )(z, codebook)
    return idxT.reshape(B, T), losssum[0, 0]


_NC, _NS = 2, 16
_NW = _NC * _NS
_CHUNK = 96


def _sc_gather_body(n, per, cb_hbm, idx_hbm, out_hbm, idx_v, buf0, buf1,
                    sem0, sem1):
    wid = lax.axis_index("s") * _NC + lax.axis_index("c")
    base = wid * per
    pltpu.sync_copy(idx_hbm.at[pl.ds(base, per)], idx_v)
    bufs = (buf0, buf1)
    sems = (sem0, sem1)
    nch = per // _CHUNK
    copies = [None] * nch
    copies[0] = pltpu.async_copy(cb_hbm.at[idx_v.at[pl.ds(0, _CHUNK)]],
                                 bufs[0], sems[0])
    for j in range(nch):
        if j + 1 < nch:
            copies[j + 1] = pltpu.async_copy(
                cb_hbm.at[idx_v.at[pl.ds((j + 1) * _CHUNK, _CHUNK)]],
                bufs[(j + 1) % 2], sems[(j + 1) % 2])
        copies[j].wait()
        pltpu.sync_copy(bufs[j % 2],
                        out_hbm.at[pl.ds(base + j * _CHUNK, _CHUNK)])


def _sc_gather(codebook, idx_flat):
    n = idx_flat.shape[0]
    K, D = codebook.shape
    per = n // _NW
    mesh = plsc.VectorSubcoreMesh(core_axis_name="c", subcore_axis_name="s")
    fn = functools.partial(
        pl.kernel,
        out_type=jax.ShapeDtypeStruct((n, D), jnp.float32),
        mesh=mesh,
        scratch_types=[
            pltpu.VMEM((per,), jnp.int32),
            pltpu.VMEM((_CHUNK, D), jnp.float32),
            pltpu.VMEM((_CHUNK, D), jnp.float32),
            pltpu.SemaphoreType.DMA,
            pltpu.SemaphoreType.DMA,
        ],
    )(functools.partial(_sc_gather_body, n, per))
    return fn(codebook, idx_flat)


def kernel(z, codebook):
    B, T, D = z.shape
    idx, losssum = _encode(z, codebook)
    loss = losssum * ((1.0 + COMMIT) / (B * T * D))
    quant = _sc_gather(codebook, idx.reshape(-1))
    return quant.reshape(B, T, D), loss, idx


# speedup vs baseline: 1.7224x; 1.0363x over previous
import functools

import jax
import jax.numpy as jnp
from jax import lax
from jax.experimental import pallas as pl
from jax.experimental.pallas import tpu as pltpu
from jax.experimental.pallas import tpu_sc as plsc

COMMIT = 0.25


def _encode_body(z_ref, cb_ref, idx_ref, loss_ref, cnorm_ref):
    b = pl.program_id(0)
    zb = z_ref[0]
    cb = cb_ref[...]
    T, D = zb.shape
    K = cb.shape[0]
    LW = 128

    @pl.when(b == 0)
    def _():
        cnorm_ref[...] = jnp.sum(cb * cb, axis=1)[None, :]

    mm = lax.dot_general(zb, cb, (((1,), (1,)), ((), ())),
                         preferred_element_type=jnp.float32)
    znorm = jnp.sum(zb * zb, axis=1, keepdims=True)
    TT = 64
    lane = lax.broadcasted_iota(jnp.int32, (TT, LW), 1)
    losspart = jnp.zeros((1, 1), jnp.float32)
    for t in range(0, T, TT):
        zn = znorm[t:t + TT, :]
        m = argk = None
        for j in range(K // LW):
            cn = cnorm_ref[:, j * LW:(j + 1) * LW]
            dj = zn - 2.0 * mm[t:t + TT, j * LW:(j + 1) * LW] + cn
            if j == 0:
                m, argk = dj, lane
            else:
                lt = dj < m
                m = jnp.where(lt, dj, m)
                argk = jnp.where(lt, lane + j * LW, argk)
        mind = jnp.min(m, axis=1, keepdims=True)
        idx = jnp.min(jnp.where(m == mind, argk, K), axis=1, keepdims=True)
        idx_ref[0, t:t + TT, :] = idx
        losspart += jnp.sum(mind).reshape(1, 1)
    mind = losspart

    @pl.when(b == 0)
    def _():
        loss_ref[...] = jnp.zeros((1, 1), jnp.float32)

    loss_ref[...] += jnp.sum(mind).reshape(1, 1)


def _encode(z, codebook):
    B, T, D = z.shape
    K = codebook.shape[0]
    idxT, losssum = pl.pallas_call(
        _encode_body,
        grid=(B,),
        in_specs=[
            pl.BlockSpec((1, T, D), lambda i: (i, 0, 0)),
            pl.BlockSpec((K, D), lambda i: (0, 0)),
        ],
        out_specs=[
            pl.BlockSpec((1, T, 1), lambda i: (i, 0, 0)),
            pl.BlockSpec((1, 1), lambda i: (0, 0)),
        ],
        out_shape=[
            jax.ShapeDtypeStruct((B, T, 1), jnp.int32),
            jax.ShapeDtypeStruct((1, 1), jnp.float32),
        ],
        scratch_shapes=[pltpu.VMEM((1, K), jnp.float32)],
    )(z, codebook)
    return idxT.reshape(B, T), losssum[0, 0]


_NC, _NS = 2, 16
_NW = _NC * _NS
_CHUNK = 96


def _sc_gather_body(n, per, cb_hbm, idx_hbm, out_hbm, idx_v, buf0, buf1,
                    sem0, sem1):
    wid = lax.axis_index("s") * _NC + lax.axis_index("c")
    base = wid * per
    pltpu.sync_copy(idx_hbm.at[pl.ds(base, per)], idx_v)
    bufs = (buf0, buf1)
    sems = (sem0, sem1)
    nch = per // _CHUNK
    copies = [None] * nch
    copies[0] = pltpu.async_copy(cb_hbm.at[idx_v.at[pl.ds(0, _CHUNK)]],
                                 bufs[0], sems[0])
    for j in range(nch):
        if j + 1 < nch:
            copies[j + 1] = pltpu.async_copy(
                cb_hbm.at[idx_v.at[pl.ds((j + 1) * _CHUNK, _CHUNK)]],
                bufs[(j + 1) % 2], sems[(j + 1) % 2])
        copies[j].wait()
        pltpu.sync_copy(bufs[j % 2],
                        out_hbm.at[pl.ds(base + j * _CHUNK, _CHUNK)])


def _sc_gather(codebook, idx_flat):
    n = idx_flat.shape[0]
    K, D = codebook.shape
    per = n // _NW
    mesh = plsc.VectorSubcoreMesh(core_axis_name="c", subcore_axis_name="s")
    fn = functools.partial(
        pl.kernel,
        out_type=jax.ShapeDtypeStruct((n, D), jnp.float32),
        mesh=mesh,
        scratch_types=[
            pltpu.VMEM((per,), jnp.int32),
            pltpu.VMEM((_CHUNK, D), jnp.float32),
            pltpu.VMEM((_CHUNK, D), jnp.float32),
            pltpu.SemaphoreType.DMA,
            pltpu.SemaphoreType.DMA,
        ],
    )(functools.partial(_sc_gather_body, n, per))
    return fn(codebook, idx_flat)


def kernel(z, codebook):
    B, T, D = z.shape
    idx, losssum = _encode(z, codebook)
    loss = losssum * ((1.0 + COMMIT) / (B * T * D))
    quant = _sc_gather(codebook, idx.reshape(-1))
    return quant.reshape(B, T, D), loss, idx


# speedup vs baseline: 1.8477x; 1.0727x over previous
import functools

import jax
import jax.numpy as jnp
from jax import lax
from jax.experimental import pallas as pl
from jax.experimental.pallas import tpu as pltpu
from jax.experimental.pallas import tpu_sc as plsc

COMMIT = 0.25


def _encode_body(z_ref, cb_ref, idx_ref, loss_ref, cnorm_ref):
    b = pl.program_id(0)
    zb = z_ref[...]
    cb = cb_ref[...]
    T, D = zb.shape
    K = cb.shape[0]
    LW = 128

    @pl.when(b == 0)
    def _():
        cnorm_ref[...] = jnp.sum(cb * cb, axis=1)[None, :]

    mm = lax.dot_general(zb, cb, (((1,), (1,)), ((), ())),
                         preferred_element_type=jnp.float32)
    znorm = jnp.sum(zb * zb, axis=1, keepdims=True)
    TT = 64
    lane = lax.broadcasted_iota(jnp.int32, (TT, LW), 1)
    losspart = jnp.zeros((1, 1), jnp.float32)
    for t in range(0, T, TT):
        zn = znorm[t:t + TT, :]
        m = argk = None
        for j in range(K // LW):
            cn = cnorm_ref[:, j * LW:(j + 1) * LW]
            dj = zn - 2.0 * mm[t:t + TT, j * LW:(j + 1) * LW] + cn
            if j == 0:
                m, argk = dj, lane
            else:
                lt = dj < m
                m = jnp.where(lt, dj, m)
                argk = jnp.where(lt, lane + j * LW, argk)
        mind = jnp.min(m, axis=1, keepdims=True)
        idx = jnp.min(jnp.where(m == mind, argk, K), axis=1, keepdims=True)
        idx_ref[0, t:t + TT, :] = idx
        losspart += jnp.sum(mind).reshape(1, 1)
    mind = losspart

    @pl.when(b == 0)
    def _():
        loss_ref[...] = jnp.zeros((1, 1), jnp.float32)

    loss_ref[...] += jnp.sum(mind).reshape(1, 1)


_TB = 1152


def _encode(zf, codebook):
    N, D = zf.shape
    K = codebook.shape[0]
    G = N // _TB
    idx3, losssum = pl.pallas_call(
        _encode_body,
        grid=(G,),
        in_specs=[
            pl.BlockSpec((_TB, D), lambda i: (i, 0)),
            pl.BlockSpec((K, D), lambda i: (0, 0)),
        ],
        out_specs=[
            pl.BlockSpec((1, _TB, 1), lambda i: (i, 0, 0)),
            pl.BlockSpec((1, 1), lambda i: (0, 0)),
        ],
        out_shape=[
            jax.ShapeDtypeStruct((G, _TB, 1), jnp.int32),
            jax.ShapeDtypeStruct((1, 1), jnp.float32),
        ],
        scratch_shapes=[pltpu.VMEM((1, K), jnp.float32)],
    )(zf, codebook)
    return idx3.reshape(N), losssum[0, 0]


_NC, _NS = 2, 16
_NW = _NC * _NS
_CHUNK = 96


def _sc_gather_body(n, per, cb_hbm, idx_hbm, out_hbm, idx_v, buf0, buf1,
                    sem0, sem1):
    wid = lax.axis_index("s") * _NC + lax.axis_index("c")
    base = wid * per
    pltpu.sync_copy(idx_hbm.at[pl.ds(base, per)], idx_v)
    bufs = (buf0, buf1)
    sems = (sem0, sem1)
    nch = per // _CHUNK
    copies = [None] * nch
    copies[0] = pltpu.async_copy(cb_hbm.at[idx_v.at[pl.ds(0, _CHUNK)]],
                                 bufs[0], sems[0])
    for j in range(nch):
        if j + 1 < nch:
            copies[j + 1] = pltpu.async_copy(
                cb_hbm.at[idx_v.at[pl.ds((j + 1) * _CHUNK, _CHUNK)]],
                bufs[(j + 1) % 2], sems[(j + 1) % 2])
        copies[j].wait()
        pltpu.sync_copy(bufs[j % 2],
                        out_hbm.at[pl.ds(base + j * _CHUNK, _CHUNK)])


def _sc_gather(codebook, idx_flat):
    n = idx_flat.shape[0]
    K, D = codebook.shape
    per = n // _NW
    mesh = plsc.VectorSubcoreMesh(core_axis_name="c", subcore_axis_name="s")
    fn = functools.partial(
        pl.kernel,
        out_type=jax.ShapeDtypeStruct((n, D), jnp.float32),
        mesh=mesh,
        scratch_types=[
            pltpu.VMEM((per,), jnp.int32),
            pltpu.VMEM((_CHUNK, D), jnp.float32),
            pltpu.VMEM((_CHUNK, D), jnp.float32),
            pltpu.SemaphoreType.DMA,
            pltpu.SemaphoreType.DMA,
        ],
    )(functools.partial(_sc_gather_body, n, per))
    return fn(codebook, idx_flat)


def kernel(z, codebook):
    B, T, D = z.shape
    idx_flat, losssum = _encode(z.reshape(-1, D), codebook)
    loss = losssum * ((1.0 + COMMIT) / (B * T * D))
    quant = _sc_gather(codebook, idx_flat)
    return quant.reshape(B, T, D), loss, idx_flat.reshape(B, T)


# speedup vs baseline: 1.9013x; 1.0290x over previous
import functools

import jax
import jax.numpy as jnp
from jax import lax
from jax.experimental import pallas as pl
from jax.experimental.pallas import tpu as pltpu
from jax.experimental.pallas import tpu_sc as plsc

COMMIT = 0.25


def _encode_body(z_ref, cb_ref, idx_ref, loss_ref, cnorm_ref):
    b = pl.program_id(0)
    zb = z_ref[...]
    cb = cb_ref[...]
    T, D = zb.shape
    K = cb.shape[0]
    LW = 128

    @pl.when(b == 0)
    def _():
        cnorm_ref[...] = jnp.sum(cb * cb, axis=1)[None, :]

    mm = lax.dot_general(zb, cb, (((1,), (1,)), ((), ())),
                         preferred_element_type=jnp.float32)
    znorm = jnp.sum(zb * zb, axis=1, keepdims=True)
    TT = 64
    lane = lax.broadcasted_iota(jnp.int32, (TT, LW), 1)
    losspart = jnp.zeros((1, 1), jnp.float32)
    for t in range(0, T, TT):
        zn = znorm[t:t + TT, :]
        m = argk = None
        for j in range(K // LW):
            cn = cnorm_ref[:, j * LW:(j + 1) * LW]
            dj = zn - 2.0 * mm[t:t + TT, j * LW:(j + 1) * LW] + cn
            if j == 0:
                m, argk = dj, lane
            else:
                lt = dj < m
                m = jnp.where(lt, dj, m)
                argk = jnp.where(lt, lane + j * LW, argk)
        mind = jnp.min(m, axis=1, keepdims=True)
        idx = jnp.min(jnp.where(m == mind, argk, K), axis=1, keepdims=True)
        idx_ref[0, t:t + TT, :] = idx
        losspart += jnp.sum(mind).reshape(1, 1)
    mind = losspart

    @pl.when(b == 0)
    def _():
        loss_ref[...] = jnp.zeros((1, 1), jnp.float32)

    loss_ref[...] += jnp.sum(mind).reshape(1, 1)


_TB = 2304


def _encode(zf, codebook):
    N, D = zf.shape
    K = codebook.shape[0]
    G = N // _TB
    idx3, losssum = pl.pallas_call(
        _encode_body,
        grid=(G,),
        in_specs=[
            pl.BlockSpec((_TB, D), lambda i: (i, 0)),
            pl.BlockSpec((K, D), lambda i: (0, 0)),
        ],
        out_specs=[
            pl.BlockSpec((1, _TB, 1), lambda i: (i, 0, 0)),
            pl.BlockSpec((1, 1), lambda i: (0, 0)),
        ],
        out_shape=[
            jax.ShapeDtypeStruct((G, _TB, 1), jnp.int32),
            jax.ShapeDtypeStruct((1, 1), jnp.float32),
        ],
        scratch_shapes=[pltpu.VMEM((1, K), jnp.float32)],
    )(zf, codebook)
    return idx3.reshape(N), losssum[0, 0]


_NC, _NS = 2, 16
_NW = _NC * _NS
_CHUNK = 96


def _sc_gather_body(n, per, cb_hbm, idx_hbm, out_hbm, idx_v, buf0, buf1,
                    sem0, sem1):
    wid = lax.axis_index("s") * _NC + lax.axis_index("c")
    base = wid * per
    pltpu.sync_copy(idx_hbm.at[pl.ds(base, per)], idx_v)
    bufs = (buf0, buf1)
    sems = (sem0, sem1)
    nch = per // _CHUNK
    copies = [None] * nch
    copies[0] = pltpu.async_copy(cb_hbm.at[idx_v.at[pl.ds(0, _CHUNK)]],
                                 bufs[0], sems[0])
    for j in range(nch):
        if j + 1 < nch:
            copies[j + 1] = pltpu.async_copy(
                cb_hbm.at[idx_v.at[pl.ds((j + 1) * _CHUNK, _CHUNK)]],
                bufs[(j + 1) % 2], sems[(j + 1) % 2])
        copies[j].wait()
        pltpu.sync_copy(bufs[j % 2],
                        out_hbm.at[pl.ds(base + j * _CHUNK, _CHUNK)])


def _sc_gather(codebook, idx_flat):
    n = idx_flat.shape[0]
    K, D = codebook.shape
    per = n // _NW
    mesh = plsc.VectorSubcoreMesh(core_axis_name="c", subcore_axis_name="s")
    fn = functools.partial(
        pl.kernel,
        out_type=jax.ShapeDtypeStruct((n, D), jnp.float32),
        mesh=mesh,
        scratch_types=[
            pltpu.VMEM((per,), jnp.int32),
            pltpu.VMEM((_CHUNK, D), jnp.float32),
            pltpu.VMEM((_CHUNK, D), jnp.float32),
            pltpu.SemaphoreType.DMA,
            pltpu.SemaphoreType.DMA,
        ],
    )(functools.partial(_sc_gather_body, n, per))
    return fn(codebook, idx_flat)


def kernel(z, codebook):
    B, T, D = z.shape
    idx_flat, losssum = _encode(z.reshape(-1, D), codebook)
    loss = losssum * ((1.0 + COMMIT) / (B * T * D))
    quant = _sc_gather(codebook, idx_flat)
    return quant.reshape(B, T, D), loss, idx_flat.reshape(B, T)


# speedup vs baseline: 1.9031x; 1.0009x over previous
import functools

import jax
import jax.numpy as jnp
from jax import lax
from jax.experimental import pallas as pl
from jax.experimental.pallas import tpu as pltpu
from jax.experimental.pallas import tpu_sc as plsc

COMMIT = 0.25


def _encode_body(z_ref, cb_ref, idx_ref, loss_ref, cnorm_ref):
    b = pl.program_id(0)
    zb = z_ref[...]
    cb = cb_ref[...]
    T, D = zb.shape
    K = cb.shape[0]
    LW = 128

    @pl.when(b == 0)
    def _():
        cnorm_ref[...] = jnp.sum(cb * cb, axis=1)[None, :]

    mm = lax.dot_general(zb, cb, (((1,), (1,)), ((), ())),
                         preferred_element_type=jnp.float32)
    znorm = jnp.sum(zb * zb, axis=1, keepdims=True)
    TT = 64
    lane = lax.broadcasted_iota(jnp.int32, (TT, LW), 1)
    losspart = jnp.zeros((1, 1), jnp.float32)
    for t in range(0, T, TT):
        zn = znorm[t:t + TT, :]
        m = argk = None
        for j in range(K // LW):
            cn = cnorm_ref[:, j * LW:(j + 1) * LW]
            dj = zn - 2.0 * mm[t:t + TT, j * LW:(j + 1) * LW] + cn
            if j == 0:
                m, argk = dj, lane
            else:
                lt = dj < m
                m = jnp.where(lt, dj, m)
                argk = jnp.where(lt, lane + j * LW, argk)
        mind = jnp.min(m, axis=1, keepdims=True)
        idx = jnp.min(jnp.where(m == mind, argk, K), axis=1, keepdims=True)
        idx_ref[0, t:t + TT, :] = idx
        losspart += jnp.sum(mind).reshape(1, 1)
    mind = losspart

    @pl.when(b == 0)
    def _():
        loss_ref[...] = jnp.zeros((1, 1), jnp.float32)

    loss_ref[...] += jnp.sum(mind).reshape(1, 1)


_TB = 4608


def _encode(zf, codebook):
    N, D = zf.shape
    K = codebook.shape[0]
    G = N // _TB
    idx3, losssum = pl.pallas_call(
        _encode_body,
        grid=(G,),
        in_specs=[
            pl.BlockSpec((_TB, D), lambda i: (i, 0)),
            pl.BlockSpec((K, D), lambda i: (0, 0)),
        ],
        out_specs=[
            pl.BlockSpec((1, _TB, 1), lambda i: (i, 0, 0)),
            pl.BlockSpec((1, 1), lambda i: (0, 0)),
        ],
        out_shape=[
            jax.ShapeDtypeStruct((G, _TB, 1), jnp.int32),
            jax.ShapeDtypeStruct((1, 1), jnp.float32),
        ],
        scratch_shapes=[pltpu.VMEM((1, K), jnp.float32)],
    )(zf, codebook)
    return idx3.reshape(N), losssum[0, 0]


_NC, _NS = 2, 16
_NW = _NC * _NS
_CHUNK = 96


def _sc_gather_body(n, per, cb_hbm, idx_hbm, out_hbm, idx_v, buf0, buf1,
                    sem0, sem1):
    wid = lax.axis_index("s") * _NC + lax.axis_index("c")
    base = wid * per
    pltpu.sync_copy(idx_hbm.at[pl.ds(base, per)], idx_v)
    bufs = (buf0, buf1)
    sems = (sem0, sem1)
    nch = per // _CHUNK
    copies = [None] * nch
    copies[0] = pltpu.async_copy(cb_hbm.at[idx_v.at[pl.ds(0, _CHUNK)]],
                                 bufs[0], sems[0])
    for j in range(nch):
        if j + 1 < nch:
            copies[j + 1] = pltpu.async_copy(
                cb_hbm.at[idx_v.at[pl.ds((j + 1) * _CHUNK, _CHUNK)]],
                bufs[(j + 1) % 2], sems[(j + 1) % 2])
        copies[j].wait()
        pltpu.sync_copy(bufs[j % 2],
                        out_hbm.at[pl.ds(base + j * _CHUNK, _CHUNK)])


def _sc_gather(codebook, idx_flat):
    n = idx_flat.shape[0]
    K, D = codebook.shape
    per = n // _NW
    mesh = plsc.VectorSubcoreMesh(core_axis_name="c", subcore_axis_name="s")
    fn = functools.partial(
        pl.kernel,
        out_type=jax.ShapeDtypeStruct((n, D), jnp.float32),
        mesh=mesh,
        scratch_types=[
            pltpu.VMEM((per,), jnp.int32),
            pltpu.VMEM((_CHUNK, D), jnp.float32),
            pltpu.VMEM((_CHUNK, D), jnp.float32),
            pltpu.SemaphoreType.DMA,
            pltpu.SemaphoreType.DMA,
        ],
    )(functools.partial(_sc_gather_body, n, per))
    return fn(codebook, idx_flat)


def kernel(z, codebook):
    B, T, D = z.shape
    idx_flat, losssum = _encode(z.reshape(-1, D), codebook)
    loss = losssum * ((1.0 + COMMIT) / (B * T * D))
    quant = _sc_gather(codebook, idx_flat)
    return quant.reshape(B, T, D), loss, idx_flat.reshape(B, T)
